# Initial kernel scaffold; baseline (speedup 1.0000x reference)
import jax
import jax.numpy as jnp
from jax.experimental import pallas as pl


def kernel(images, masks, colors):
    raise NotImplementedError("write your pallas kernel here")


# speedup vs baseline: 1.2677x; 1.2677x over previous
import math

import jax
import jax.numpy as jnp
from jax.experimental import pallas as pl
from jax.experimental.pallas import tpu as pltpu

_KDE_BW = 4.0
_NBIN = 128
_CONST1 = (2.0 * math.pi * _KDE_BW) ** (-0.5)
_CONST2 = 2.0 * _KDE_BW
_LOG2E = 1.4426950408889634
_ALPHA = _LOG2E / _CONST2
_SQRT_ALPHA = math.sqrt(_ALPHA)
_NEG_BIG = -1.0e30

_NGRP = 16
_ROWS_PER_ITER = 8


def _kde_kernel(x_ref, m_ref, c1_ref, c2_ref, o_ref):
    r_rows = x_ref.shape[1]

    c1 = [c1_ref[t] for t in range(_NGRP)]
    c2 = [c2_ref[t] for t in range(_NGRP)]

    def body(j, carry):
        accs, macc = carry
        base = j * _ROWS_PER_ITER
        x8 = x_ref[0, pl.ds(base, _ROWS_PER_ITER), :]
        m8 = m_ref[0, pl.ds(base, _ROWS_PER_ITER), :]
        accs = list(accs)
        for s in range(_ROWS_PER_ITER):
            x = x8[s : s + 1, :]
            m = m8[s : s + 1, :]
            xs = x * _SQRT_ALPHA
            xx = xs * xs
            mb = (m - 1.0) * (-_NEG_BIG)
            bias = mb - xx
            xb = jnp.broadcast_to(xs, (8, 128))
            bb = jnp.broadcast_to(bias, (8, 128))
            for t in range(_NGRP):
                tt = xb * c1[t] + (bb - c2[t])
                accs[t] = accs[t] + jnp.exp2(tt)
            macc = macc + m
        return tuple(accs), macc

    accs0 = tuple(jnp.zeros((8, 128), jnp.float32) for _ in range(_NGRP))
    macc0 = jnp.zeros((1, 128), jnp.float32)
    accs, macc = jax.lax.fori_loop(
        0, r_rows // _ROWS_PER_ITER, body, (accs0, macc0)
    )

    stacked = jnp.concatenate(accs, axis=0)
    ones = jnp.ones((1, 128), jnp.float32)
    p_row = jax.lax.dot_general(
        ones, stacked, (((1,), (1,)), ((), ())),
        preferred_element_type=jnp.float32,
    )
    msum = jnp.sum(macc, axis=1, keepdims=True)
    inv = jnp.where(msum != 0.0, 1.0 / msum, 1.0)
    o_ref[0] = p_row * (inv * _CONST1)


def kernel(images, masks, colors):
    B, C, H, W = images.shape
    P = H * W
    R = P // 128
    x3 = images.reshape(B * C, R, 128)
    m3 = masks.reshape(B, R, 128)

    csc = (colors * _SQRT_ALPHA).reshape(_NGRP, 8, 1)
    c1b = jnp.broadcast_to(2.0 * csc, (_NGRP, 8, 128))
    c2b = jnp.broadcast_to(csc * csc, (_NGRP, 8, 128))

    out = pl.pallas_call(
        _kde_kernel,
        grid=(B * C,),
        in_specs=[
            pl.BlockSpec((1, R, 128), lambda i: (i, 0, 0)),
            pl.BlockSpec((1, R, 128), lambda i: (i // C, 0, 0)),
            pl.BlockSpec((_NGRP, 8, 128), lambda i: (0, 0, 0)),
            pl.BlockSpec((_NGRP, 8, 128), lambda i: (0, 0, 0)),
        ],
        out_specs=pl.BlockSpec((1, 1, 128), lambda i: (i, 0, 0)),
        out_shape=jax.ShapeDtypeStruct((B * C, 1, 128), jnp.float32),
        compiler_params=pltpu.CompilerParams(
            dimension_semantics=("parallel",)
        ),
    )(x3, m3, c1b, c2b)
    return out.reshape(B, C, _NBIN)


# speedup vs baseline: 1.3427x; 1.0592x over previous
import math

import jax
import jax.numpy as jnp
from jax.experimental import pallas as pl
from jax.experimental.pallas import tpu as pltpu

_KDE_BW = 4.0
_NBIN = 128
_CONST1 = (2.0 * math.pi * _KDE_BW) ** (-0.5)
_CONST2 = 2.0 * _KDE_BW
_LOG2E = 1.4426950408889634
_ALPHA = _LOG2E / _CONST2
_SQRT_ALPHA = math.sqrt(_ALPHA)
_NEG_BIG = -1.0e30

_NGRP = 16
_ROWS_PER_ITER = 8


def _kde_kernel(x_ref, m_ref, c1_ref, o_ref):
    r_rows = x_ref.shape[1]

    c1 = [c1_ref[t] for t in range(_NGRP)]

    def body(j, carry):
        accs, macc = carry
        base = j * _ROWS_PER_ITER
        x8 = x_ref[0, pl.ds(base, _ROWS_PER_ITER), :]
        m8 = m_ref[0, pl.ds(base, _ROWS_PER_ITER), :]
        accs = list(accs)
        for s in range(_ROWS_PER_ITER):
            x = x8[s : s + 1, :]
            m = m8[s : s + 1, :]
            xs = x * _SQRT_ALPHA
            mb = (m - 1.0) * (-_NEG_BIG)
            xb = jnp.broadcast_to(xs, (8, 128))
            bb = jnp.broadcast_to(mb, (8, 128))
            for t in range(_NGRP):
                d = xb - c1[t]
                tt = bb - d * d
                accs[t] = accs[t] + jnp.exp2(tt)
            macc = macc + m
        return tuple(accs), macc

    accs0 = tuple(jnp.zeros((8, 128), jnp.float32) for _ in range(_NGRP))
    macc0 = jnp.zeros((1, 128), jnp.float32)
    accs, macc = jax.lax.fori_loop(
        0, r_rows // _ROWS_PER_ITER, body, (accs0, macc0)
    )

    stacked = jnp.concatenate(accs, axis=0)
    ones = jnp.ones((1, 128), jnp.float32)
    p_row = jax.lax.dot_general(
        ones, stacked, (((1,), (1,)), ((), ())),
        preferred_element_type=jnp.float32,
    )
    msum = jnp.sum(macc, axis=1, keepdims=True)
    inv = jnp.where(msum != 0.0, 1.0 / msum, 1.0)
    o_ref[0] = p_row * (inv * _CONST1)


def kernel(images, masks, colors):
    B, C, H, W = images.shape
    P = H * W
    R = P // 128
    x3 = images.reshape(B * C, R, 128)
    m3 = masks.reshape(B, R, 128)

    csc = (colors * _SQRT_ALPHA).reshape(_NGRP, 8, 1)
    c1b = jnp.broadcast_to(csc, (_NGRP, 8, 128))

    out = pl.pallas_call(
        _kde_kernel,
        grid=(B * C,),
        in_specs=[
            pl.BlockSpec((1, R, 128), lambda i: (i, 0, 0)),
            pl.BlockSpec((1, R, 128), lambda i: (i // C, 0, 0)),
            pl.BlockSpec((_NGRP, 8, 128), lambda i: (0, 0, 0)),
        ],
        out_specs=pl.BlockSpec((1, 1, 128), lambda i: (i, 0, 0)),
        out_shape=jax.ShapeDtypeStruct((B * C, 1, 128), jnp.float32),
        compiler_params=pltpu.CompilerParams(
            dimension_semantics=("parallel",)
        ),
    )(x3, m3, c1b)
    return out.reshape(B, C, _NBIN)
